# Initial kernel scaffold; baseline (speedup 1.0000x reference)
import jax
import jax.numpy as jnp
from jax.experimental import pallas as pl


def kernel(features, edge_index, W, b):
    raise NotImplementedError("write your pallas kernel here")


# speedup vs baseline: 7.6071x; 7.6071x over previous
import functools

import jax
import jax.numpy as jnp
from jax import lax
from jax.experimental import pallas as pl
from jax.experimental.pallas import tpu as pltpu
from jax.experimental.pallas import tpu_sc as plsc

N_NODES = 10000
N_EDGES = 320000
D_FEAT = 128
D_AUG = D_FEAT + 16

NUM_CORES = 2
NUM_SUBCORES = 16
NUM_WORKERS = NUM_CORES * NUM_SUBCORES
EDGES_PER_WORKER = N_EDGES // NUM_WORKERS
CHUNK = 80
NUM_CHUNKS = EDGES_PER_WORKER // CHUNK
ROWS_PER_TILE = N_NODES // NUM_SUBCORES
ZROWS = 25


def _sc_body(feat_hbm, src_hbm, dst_hbm, out_hbm,
             acc_sh, src_v, dst_v, rows_v, zeros_v, sem):
    cid = lax.axis_index("c")
    sid = lax.axis_index("s")
    wid = cid * NUM_SUBCORES + sid

    zvec = jnp.zeros((16,), jnp.float32)

    def _zrow(i, _):
        for k in range(D_AUG // 16):
            zeros_v[i, pl.ds(k * 16, 16)] = zvec
        return 0

    lax.fori_loop(0, ZROWS, _zrow, 0)
    row0 = sid * ROWS_PER_TILE

    def _zcopy(q, _):
        pltpu.sync_copy(zeros_v, acc_sh.at[pl.ds(row0 + q * ZROWS, ZROWS)])
        return 0

    lax.fori_loop(0, ROWS_PER_TILE // ZROWS, _zcopy, 0)

    pltpu.sync_copy(src_hbm.at[wid], src_v)
    pltpu.sync_copy(dst_hbm.at[wid], dst_v)

    plsc.subcore_barrier()

    def _step(j, _):
        pltpu.async_copy(feat_hbm.at[src_v.at[j]], rows_v, sem).wait()
        pltpu.sync_copy(rows_v, acc_sh.at[dst_v.at[j]], add=True)
        return 0

    lax.fori_loop(0, NUM_CHUNKS, _step, 0)

    plsc.subcore_barrier()

    out_base = cid * N_NODES + sid * ROWS_PER_TILE
    pltpu.sync_copy(acc_sh.at[pl.ds(sid * ROWS_PER_TILE, ROWS_PER_TILE)],
                    out_hbm.at[pl.ds(out_base, ROWS_PER_TILE)])


@functools.lru_cache(maxsize=1)
def _sc_agg():
    return functools.partial(
        pl.kernel,
        out_type=jax.ShapeDtypeStruct((NUM_CORES * N_NODES, D_AUG), jnp.float32),
        mesh=plsc.VectorSubcoreMesh(core_axis_name="c", subcore_axis_name="s"),
        scratch_types=[
            pltpu.VMEM_SHARED((N_NODES, D_AUG), jnp.float32),
            pltpu.VMEM((NUM_CHUNKS, CHUNK), jnp.int32),
            pltpu.VMEM((NUM_CHUNKS, CHUNK), jnp.int32),
            pltpu.VMEM((CHUNK, D_AUG), jnp.float32),
            pltpu.VMEM((ZROWS, D_AUG), jnp.float32),
            pltpu.SemaphoreType.DMA,
        ],
        compiler_params=pltpu.CompilerParams(use_tc_tiling_on_sc=False),
    )(_sc_body)


def _tc_body(p_ref, f_ref, wt_ref, b_ref, o_ref):
    s = p_ref[0] + p_ref[1]
    feat_sum = s[:, :D_FEAT]
    deg = s[:, D_FEAT:D_FEAT + 1]
    mean = feat_sum / jnp.maximum(deg, 1.0)
    h = jnp.where(deg > 0.0, mean, f_ref[...])
    y = jnp.dot(h, wt_ref[...], preferred_element_type=jnp.float32)
    o_ref[...] = jnp.maximum(y + b_ref[...], 0.0)


_BR = 1000


def _tc_finish(partials, features, wt, b2):
    grid = (N_NODES // _BR,)
    return pl.pallas_call(
        _tc_body,
        grid=grid,
        in_specs=[
            pl.BlockSpec((NUM_CORES, _BR, D_AUG), lambda i: (0, i, 0)),
            pl.BlockSpec((_BR, D_FEAT), lambda i: (i, 0)),
            pl.BlockSpec((D_FEAT, D_FEAT), lambda i: (0, 0)),
            pl.BlockSpec((1, D_FEAT), lambda i: (0, 0)),
        ],
        out_specs=pl.BlockSpec((_BR, D_FEAT), lambda i: (i, 0)),
        out_shape=jax.ShapeDtypeStruct((N_NODES, D_FEAT), jnp.float32),
    )(partials, features, wt, b2)


def kernel(features, edge_index, W, b):
    src = edge_index[0].astype(jnp.int32).reshape(NUM_WORKERS, NUM_CHUNKS, CHUNK)
    dst = edge_index[1].astype(jnp.int32).reshape(NUM_WORKERS, NUM_CHUNKS, CHUNK)
    feat_aug = jnp.concatenate(
        [features, jnp.ones((N_NODES, 16), jnp.float32)], axis=1)
    partials = _sc_agg()(feat_aug, src, dst)
    partials = partials.reshape(NUM_CORES, N_NODES, D_AUG)
    return _tc_finish(partials, features, W.T, b.reshape(1, D_FEAT))


# speedup vs baseline: 9.2935x; 1.2217x over previous
import functools

import jax
import jax.numpy as jnp
from jax import lax
from jax.experimental import pallas as pl
from jax.experimental.pallas import tpu as pltpu
from jax.experimental.pallas import tpu_sc as plsc

N_NODES = 10000
N_EDGES = 320000
D_FEAT = 128
D_AUG = D_FEAT + 16

NUM_CORES = 2
NUM_SUBCORES = 16
NUM_WORKERS = NUM_CORES * NUM_SUBCORES
EDGES_PER_WORKER = N_EDGES // NUM_WORKERS
CHUNK = 40
NUM_CHUNKS = EDGES_PER_WORKER // CHUNK
ROWS_PER_TILE = N_NODES // NUM_SUBCORES
ZROWS = 25


def _sc_body(feat_hbm, src_hbm, dst_hbm, out_hbm,
             acc_sh, src_v, dst_v, rows0, rows1, zeros_v, sem0, sem1):
    cid = lax.axis_index("c")
    sid = lax.axis_index("s")
    wid = cid * NUM_SUBCORES + sid

    zvec = jnp.zeros((16,), jnp.float32)

    def _zrow(i, _):
        for k in range(D_AUG // 16):
            zeros_v[i, pl.ds(k * 16, 16)] = zvec
        return 0

    lax.fori_loop(0, ZROWS, _zrow, 0)
    row0 = sid * ROWS_PER_TILE

    def _zcopy(q, _):
        pltpu.sync_copy(zeros_v, acc_sh.at[pl.ds(row0 + q * ZROWS, ZROWS)])
        return 0

    lax.fori_loop(0, ROWS_PER_TILE // ZROWS, _zcopy, 0)

    pltpu.sync_copy(src_hbm.at[wid], src_v)
    pltpu.sync_copy(dst_hbm.at[wid], dst_v)

    plsc.subcore_barrier()

    pltpu.async_copy(feat_hbm.at[src_v.at[0]], rows0, sem0)

    def _step(i, _):
        j = 2 * i
        pltpu.async_copy(feat_hbm.at[src_v.at[j + 1]], rows1, sem1)
        pltpu.make_async_copy(feat_hbm.at[src_v.at[j]], rows0, sem0).wait()
        pltpu.sync_copy(rows0, acc_sh.at[dst_v.at[j]], add=True)

        @pl.when(i < NUM_CHUNKS // 2 - 1)
        def _():
            pltpu.async_copy(feat_hbm.at[src_v.at[j + 2]], rows0, sem0)

        pltpu.make_async_copy(feat_hbm.at[src_v.at[j + 1]], rows1, sem1).wait()
        pltpu.sync_copy(rows1, acc_sh.at[dst_v.at[j + 1]], add=True)
        return 0

    lax.fori_loop(0, NUM_CHUNKS // 2, _step, 0)

    plsc.subcore_barrier()

    out_base = cid * N_NODES + sid * ROWS_PER_TILE
    pltpu.sync_copy(acc_sh.at[pl.ds(sid * ROWS_PER_TILE, ROWS_PER_TILE)],
                    out_hbm.at[pl.ds(out_base, ROWS_PER_TILE)])


@functools.lru_cache(maxsize=1)
def _sc_agg():
    return functools.partial(
        pl.kernel,
        out_type=jax.ShapeDtypeStruct((NUM_CORES * N_NODES, D_AUG), jnp.float32),
        mesh=plsc.VectorSubcoreMesh(core_axis_name="c", subcore_axis_name="s"),
        scratch_types=[
            pltpu.VMEM_SHARED((N_NODES, D_AUG), jnp.float32),
            pltpu.VMEM((NUM_CHUNKS, CHUNK), jnp.int32),
            pltpu.VMEM((NUM_CHUNKS, CHUNK), jnp.int32),
            pltpu.VMEM((CHUNK, D_AUG), jnp.float32),
            pltpu.VMEM((CHUNK, D_AUG), jnp.float32),
            pltpu.VMEM((ZROWS, D_AUG), jnp.float32),
            pltpu.SemaphoreType.DMA,
            pltpu.SemaphoreType.DMA,
        ],
        compiler_params=pltpu.CompilerParams(use_tc_tiling_on_sc=False),
    )(_sc_body)


def _tc_body(p_ref, f_ref, wt_ref, b_ref, o_ref):
    s = p_ref[0] + p_ref[1]
    feat_sum = s[:, :D_FEAT]
    deg = s[:, D_FEAT:D_FEAT + 1]
    mean = feat_sum / jnp.maximum(deg, 1.0)
    h = jnp.where(deg > 0.0, mean, f_ref[...])
    y = jnp.dot(h, wt_ref[...], preferred_element_type=jnp.float32)
    o_ref[...] = jnp.maximum(y + b_ref[...], 0.0)


_BR = 1000


def _tc_finish(partials, features, wt, b2):
    grid = (N_NODES // _BR,)
    return pl.pallas_call(
        _tc_body,
        grid=grid,
        in_specs=[
            pl.BlockSpec((NUM_CORES, _BR, D_AUG), lambda i: (0, i, 0)),
            pl.BlockSpec((_BR, D_FEAT), lambda i: (i, 0)),
            pl.BlockSpec((D_FEAT, D_FEAT), lambda i: (0, 0)),
            pl.BlockSpec((1, D_FEAT), lambda i: (0, 0)),
        ],
        out_specs=pl.BlockSpec((_BR, D_FEAT), lambda i: (i, 0)),
        out_shape=jax.ShapeDtypeStruct((N_NODES, D_FEAT), jnp.float32),
    )(partials, features, wt, b2)


def kernel(features, edge_index, W, b):
    src = edge_index[0].astype(jnp.int32).reshape(NUM_WORKERS, NUM_CHUNKS, CHUNK)
    dst = edge_index[1].astype(jnp.int32).reshape(NUM_WORKERS, NUM_CHUNKS, CHUNK)
    feat_aug = jnp.concatenate(
        [features, jnp.ones((N_NODES, 16), jnp.float32)], axis=1)
    partials = _sc_agg()(feat_aug, src, dst)
    partials = partials.reshape(NUM_CORES, N_NODES, D_AUG)
    return _tc_finish(partials, features, W.T, b.reshape(1, D_FEAT))


# speedup vs baseline: 10.6924x; 1.1505x over previous
import functools

import jax
import jax.numpy as jnp
from jax import lax
from jax.experimental import pallas as pl
from jax.experimental.pallas import tpu as pltpu
from jax.experimental.pallas import tpu_sc as plsc

N_NODES = 10000
N_EDGES = 320000
D_FEAT = 128
D_DEG = 16

NUM_CORES = 2
NUM_SUBCORES = 16
NUM_WORKERS = NUM_CORES * NUM_SUBCORES
EDGES_PER_WORKER = N_EDGES // NUM_WORKERS
CHUNK = 40
NUM_CHUNKS = EDGES_PER_WORKER // CHUNK
ROWS_PER_TILE = N_NODES // NUM_SUBCORES
ZROWS = 25


def _sc_body(feat_hbm, src_hbm, dst_hbm, outf_hbm, outd_hbm,
             accf_sh, accd_sh, src_v, dst_v, rows0, rows1, ones_v,
             zerof_v, zerod_v, g0, g1, s0, s1, sd):
    cid = lax.axis_index("c")
    sid = lax.axis_index("s")
    wid = cid * NUM_SUBCORES + sid

    zvec = jnp.zeros((16,), jnp.float32)
    ovec = jnp.ones((16,), jnp.float32)

    def _zrow(i, _):
        for k in range(D_FEAT // 16):
            zerof_v[i, pl.ds(k * 16, 16)] = zvec
        zerod_v[i, pl.ds(0, 16)] = zvec
        return 0

    lax.fori_loop(0, ZROWS, _zrow, 0)

    def _orow(i, _):
        ones_v[i, pl.ds(0, 16)] = ovec
        return 0

    lax.fori_loop(0, CHUNK, _orow, 0)

    row0 = sid * ROWS_PER_TILE

    def _zcopy(q, _):
        pltpu.sync_copy(zerof_v, accf_sh.at[pl.ds(row0 + q * ZROWS, ZROWS)])
        pltpu.sync_copy(zerod_v, accd_sh.at[pl.ds(row0 + q * ZROWS, ZROWS)])
        return 0

    lax.fori_loop(0, ROWS_PER_TILE // ZROWS, _zcopy, 0)

    pltpu.sync_copy(src_hbm.at[wid], src_v)
    pltpu.sync_copy(dst_hbm.at[wid], dst_v)

    plsc.subcore_barrier()

    pltpu.async_copy(feat_hbm.at[src_v.at[0]], rows0, g0)
    pltpu.async_copy(feat_hbm.at[src_v.at[1]], rows1, g1)

    def _step(i, _):
        j = 2 * i
        pltpu.make_async_copy(feat_hbm.at[src_v.at[j]], rows0, g0).wait()
        pltpu.async_copy(ones_v, accd_sh.at[dst_v.at[j]], sd, add=True)
        pltpu.sync_copy(rows0, accf_sh.at[dst_v.at[j]], add=True)

        @pl.when(i < NUM_CHUNKS // 2 - 1)
        def _():
            pltpu.async_copy(feat_hbm.at[src_v.at[j + 2]], rows0, g0)

        pltpu.make_async_copy(feat_hbm.at[src_v.at[j + 1]], rows1, g1).wait()
        pltpu.async_copy(ones_v, accd_sh.at[dst_v.at[j + 1]], sd, add=True)
        pltpu.sync_copy(rows1, accf_sh.at[dst_v.at[j + 1]], add=True)

        @pl.when(i < NUM_CHUNKS // 2 - 1)
        def _():
            pltpu.async_copy(feat_hbm.at[src_v.at[j + 3]], rows1, g1)

        return 0

    lax.fori_loop(0, NUM_CHUNKS // 2, _step, 0)

    pltpu.make_async_copy(outd_hbm.at[pl.ds(0, N_NODES)], accd_sh, sd).wait()

    plsc.subcore_barrier()

    out_base = cid * N_NODES + sid * ROWS_PER_TILE
    pltpu.sync_copy(accf_sh.at[pl.ds(row0, ROWS_PER_TILE)],
                    outf_hbm.at[pl.ds(out_base, ROWS_PER_TILE)])
    pltpu.sync_copy(accd_sh.at[pl.ds(row0, ROWS_PER_TILE)],
                    outd_hbm.at[pl.ds(out_base, ROWS_PER_TILE)])


@functools.lru_cache(maxsize=1)
def _sc_agg():
    return functools.partial(
        pl.kernel,
        out_type=(
            jax.ShapeDtypeStruct((NUM_CORES * N_NODES, D_FEAT), jnp.float32),
            jax.ShapeDtypeStruct((NUM_CORES * N_NODES, D_DEG), jnp.float32),
        ),
        mesh=plsc.VectorSubcoreMesh(core_axis_name="c", subcore_axis_name="s"),
        scratch_types=[
            pltpu.VMEM_SHARED((N_NODES, D_FEAT), jnp.float32),
            pltpu.VMEM_SHARED((N_NODES, D_DEG), jnp.float32),
            pltpu.VMEM((NUM_CHUNKS, CHUNK), jnp.int32),
            pltpu.VMEM((NUM_CHUNKS, CHUNK), jnp.int32),
            pltpu.VMEM((CHUNK, D_FEAT), jnp.float32),
            pltpu.VMEM((CHUNK, D_FEAT), jnp.float32),
            pltpu.VMEM((CHUNK, D_DEG), jnp.float32),
            pltpu.VMEM((ZROWS, D_FEAT), jnp.float32),
            pltpu.VMEM((ZROWS, D_DEG), jnp.float32),
            pltpu.SemaphoreType.DMA,
            pltpu.SemaphoreType.DMA,
            pltpu.SemaphoreType.DMA,
            pltpu.SemaphoreType.DMA,
            pltpu.SemaphoreType.DMA,
        ],
        compiler_params=pltpu.CompilerParams(use_tc_tiling_on_sc=False),
    )(_sc_body)


def _tc_body(pf_ref, pd_ref, f_ref, wt_ref, b_ref, o_ref):
    feat_sum = pf_ref[0] + pf_ref[1]
    deg = pd_ref[0] + pd_ref[1]
    mean = feat_sum / jnp.maximum(deg, 1.0)
    h = jnp.where(deg > 0.0, mean, f_ref[...])
    y = jnp.dot(h, wt_ref[...], preferred_element_type=jnp.float32)
    o_ref[...] = jnp.maximum(y + b_ref[...], 0.0)


_BR = 1000


def _tc_finish(pfeat, pdeg, features, wt, b2):
    grid = (N_NODES // _BR,)
    return pl.pallas_call(
        _tc_body,
        grid=grid,
        in_specs=[
            pl.BlockSpec((NUM_CORES, _BR, D_FEAT), lambda i: (0, i, 0)),
            pl.BlockSpec((NUM_CORES, _BR, 1), lambda i: (0, i, 0)),
            pl.BlockSpec((_BR, D_FEAT), lambda i: (i, 0)),
            pl.BlockSpec((D_FEAT, D_FEAT), lambda i: (0, 0)),
            pl.BlockSpec((1, D_FEAT), lambda i: (0, 0)),
        ],
        out_specs=pl.BlockSpec((_BR, D_FEAT), lambda i: (i, 0)),
        out_shape=jax.ShapeDtypeStruct((N_NODES, D_FEAT), jnp.float32),
    )(pfeat, pdeg, features, wt, b2)


def kernel(features, edge_index, W, b):
    src = edge_index[0].astype(jnp.int32).reshape(NUM_WORKERS, NUM_CHUNKS, CHUNK)
    dst = edge_index[1].astype(jnp.int32).reshape(NUM_WORKERS, NUM_CHUNKS, CHUNK)
    pfeat, pdeg = _sc_agg()(features, src, dst)
    pfeat = pfeat.reshape(NUM_CORES, N_NODES, D_FEAT)
    pdeg = pdeg.reshape(NUM_CORES, N_NODES, D_DEG)[:, :, :1]
    return _tc_finish(pfeat, pdeg, features, W.T, b.reshape(1, D_FEAT))
